# Initial kernel scaffold; baseline (speedup 1.0000x reference)
import jax
import jax.numpy as jnp
from jax.experimental import pallas as pl


def kernel(input, non_zero, W, a):
    raise NotImplementedError("write your pallas kernel here")


# speedup vs baseline: 6.5800x; 6.5800x over previous
import functools

import jax
import jax.numpy as jnp
from jax import lax
from jax.experimental import pallas as pl
from jax.experimental.pallas import tpu as pltpu
from jax.experimental.pallas import tpu_sc as plsc

N_NODES = 10000
N_PAD = 10240
N_EDGES = 320000
F = 128

NUM_CORES = 2
NUM_SUBCORES = 16
NW = NUM_CORES * NUM_SUBCORES
EDGES_PER_W = N_EDGES // NW
CHUNK = 80
N_CHUNKS = EDGES_PER_W // CHUNK
ROWS_PER_TILE = N_PAD // NUM_SUBCORES

BN = 256
GRID = N_PAD // BN


def _hst_body(x_ref, w_ref, a_ref, h_ref, st_ref):
    h = jnp.dot(x_ref[...], w_ref[...], preferred_element_type=jnp.float32)
    h_ref[...] = h
    a1 = a_ref[0, :]
    a2 = a_ref[1, :]
    s = jnp.dot(h, a1, preferred_element_type=jnp.float32)
    t = jnp.dot(h, a2, preferred_element_type=jnp.float32)
    pad = jnp.zeros((6, BN), jnp.float32)
    st_ref[...] = jnp.concatenate([s[None, :], t[None, :], pad], axis=0)


def _stage1(x_pad, W, a2x128):
    return pl.pallas_call(
        _hst_body,
        grid=(GRID,),
        in_specs=[
            pl.BlockSpec((BN, F), lambda i: (i, 0)),
            pl.BlockSpec((F, F), lambda i: (0, 0)),
            pl.BlockSpec((2, F), lambda i: (0, 0)),
        ],
        out_specs=[
            pl.BlockSpec((BN, F), lambda i: (i, 0)),
            pl.BlockSpec((8, BN), lambda i: (0, i)),
        ],
        out_shape=[
            jax.ShapeDtypeStruct((N_PAD, F), jnp.float32),
            jax.ShapeDtypeStruct((8, N_PAD), jnp.float32),
        ],
    )(x_pad, W, a2x128)


def _sc_body(h_hbm, s_hbm, t_hbm, src_hbm, dst_hbm,
             part_hbm, ee_hbm,
             acc, s_tab, t_tab, ee_buf, src_v, dst_v, rows, sem):
    cid = lax.axis_index("c")
    sid = lax.axis_index("s")
    wid = sid * NUM_CORES + cid

    pltpu.sync_copy(s_hbm, s_tab)
    pltpu.sync_copy(t_hbm, t_tab)

    zeros16 = jnp.zeros((16,), jnp.float32)

    def _zrow(j, carry):
        for r in range(F // 16):
            rows[j, pl.ds(r * 16, 16)] = zeros16
        return carry

    lax.fori_loop(0, CHUNK, _zrow, 0)
    for k in range(ROWS_PER_TILE // CHUNK):
        pltpu.sync_copy(rows, acc.at[pl.ds(sid * ROWS_PER_TILE + k * CHUNK, CHUNK)])
    plsc.subcore_barrier()

    base_e = wid * EDGES_PER_W

    def _chunk(c, carry):
        off = base_e + c * CHUNK
        pltpu.sync_copy(src_hbm.at[pl.ds(off, CHUNK)], src_v)
        pltpu.sync_copy(dst_hbm.at[pl.ds(off, CHUNK)], dst_v)
        pltpu.async_copy(h_hbm.at[dst_v], rows, sem).wait()

        for k in range(CHUNK // 16):
            si = src_v[pl.ds(k * 16, 16)]
            di = dst_v[pl.ds(k * 16, 16)]
            sv = plsc.load_gather(s_tab, [si])
            tv = plsc.load_gather(t_tab, [di])
            e = sv + tv
            e = jnp.maximum(e, 0.2 * e)
            ev = jnp.exp(-e)
            ee_buf[pl.ds(c * CHUNK + k * 16, 16)] = ev
            for jj in range(16):
                eb = lax.gather(
                    ev, jnp.full((16, 1), jj, jnp.int32),
                    dimension_numbers=lax.GatherDimensionNumbers(
                        offset_dims=(), collapsed_slice_dims=(0,),
                        start_index_map=(0,)),
                    slice_sizes=(1,),
                    mode=lax.GatherScatterMode.PROMISE_IN_BOUNDS)
                j = k * 16 + jj
                for r in range(F // 16):
                    rows[j, pl.ds(r * 16, 16)] = rows[j, pl.ds(r * 16, 16)] * eb

        pltpu.sync_copy(rows, acc.at[src_v], add=True)
        return carry

    lax.fori_loop(0, N_CHUNKS, _chunk, 0)
    plsc.subcore_barrier()

    row0 = sid * ROWS_PER_TILE
    pltpu.sync_copy(acc.at[pl.ds(row0, ROWS_PER_TILE)],
                    part_hbm.at[cid, pl.ds(row0, ROWS_PER_TILE)])
    pltpu.sync_copy(ee_buf, ee_hbm.at[pl.ds(base_e, EDGES_PER_W)])


def _stage2(h, s, t, src, dst):
    mesh = plsc.VectorSubcoreMesh(
        core_axis_name="c", subcore_axis_name="s",
        num_cores=NUM_CORES, num_subcores=NUM_SUBCORES)
    f = functools.partial(
        pl.kernel,
        out_type=[
            jax.ShapeDtypeStruct((NUM_CORES, N_PAD, F), jnp.float32),
            jax.ShapeDtypeStruct((N_EDGES,), jnp.float32),
        ],
        mesh=mesh,
        compiler_params=pltpu.CompilerParams(needs_layout_passes=False),
        scratch_types=[
            pltpu.VMEM_SHARED((N_PAD, F), jnp.float32),
            pltpu.VMEM((N_PAD,), jnp.float32),
            pltpu.VMEM((N_PAD,), jnp.float32),
            pltpu.VMEM((EDGES_PER_W,), jnp.float32),
            pltpu.VMEM((CHUNK,), jnp.int32),
            pltpu.VMEM((CHUNK,), jnp.int32),
            pltpu.VMEM((CHUNK, F), jnp.float32),
            pltpu.SemaphoreType.DMA,
        ],
    )(_sc_body)
    return f(h, s, t, src, dst)


def _elu_body(p_ref, o_ref):
    z = p_ref[0] + p_ref[1]
    o_ref[...] = jnp.where(z > 0, z, jnp.exp(z) - 1.0)


def _stage3(part):
    return pl.pallas_call(
        _elu_body,
        grid=(GRID,),
        in_specs=[pl.BlockSpec((2, BN, F), lambda i: (0, i, 0))],
        out_specs=pl.BlockSpec((BN, F), lambda i: (i, 0)),
        out_shape=jax.ShapeDtypeStruct((N_PAD, F), jnp.float32),
    )(part)


def kernel(input, non_zero, W, a):
    x = jnp.asarray(input, jnp.float32)
    x_pad = jnp.pad(x, ((0, N_PAD - N_NODES), (0, 0)))
    a2 = jnp.asarray(a, jnp.float32).reshape(2, F)
    src = jnp.asarray(non_zero[0], jnp.int32)
    dst = jnp.asarray(non_zero[1], jnp.int32)

    h, st = _stage1(x_pad, jnp.asarray(W, jnp.float32), a2)
    s = st[0]
    t = st[1]

    part, ee = _stage2(h, s, t, src, dst)
    out_pad = _stage3(part)
    return (out_pad[:N_NODES], ee)


# speedup vs baseline: 9.1487x; 1.3904x over previous
import functools

import jax
import jax.numpy as jnp
from jax import lax
from jax.experimental import pallas as pl
from jax.experimental.pallas import tpu as pltpu
from jax.experimental.pallas import tpu_sc as plsc

N_NODES = 10000
N_PAD = 10240
N_EDGES = 320000
F = 128
FH = F // 2

NUM_CORES = 2
NUM_SUBCORES = 16
EDGES_PER_TILE = N_EDGES // NUM_SUBCORES
CHUNK = 80
N_CHUNKS = EDGES_PER_TILE // CHUNK
HALF_CHUNKS = N_CHUNKS // 2
EHALF = EDGES_PER_TILE // 2
ROWS_PER_TILE = N_PAD // NUM_SUBCORES

BN = 256
GRID = N_PAD // BN


def _hst_body(x_ref, w_ref, a_ref, h_ref, st_ref):
    h = jnp.dot(x_ref[...], w_ref[...], preferred_element_type=jnp.float32)
    h_ref[...] = h
    a1 = a_ref[0, :]
    a2 = a_ref[1, :]
    s = jnp.dot(h, a1, preferred_element_type=jnp.float32)
    t = jnp.dot(h, a2, preferred_element_type=jnp.float32)
    pad = jnp.zeros((6, BN), jnp.float32)
    st_ref[...] = jnp.concatenate([s[None, :], t[None, :], pad], axis=0)


def _stage1(x_pad, W, a2x128):
    return pl.pallas_call(
        _hst_body,
        grid=(GRID,),
        in_specs=[
            pl.BlockSpec((BN, F), lambda i: (i, 0)),
            pl.BlockSpec((F, F), lambda i: (0, 0)),
            pl.BlockSpec((2, F), lambda i: (0, 0)),
        ],
        out_specs=[
            pl.BlockSpec((BN, F), lambda i: (i, 0)),
            pl.BlockSpec((8, BN), lambda i: (0, i)),
        ],
        out_shape=[
            jax.ShapeDtypeStruct((N_PAD, F), jnp.float32),
            jax.ShapeDtypeStruct((8, N_PAD), jnp.float32),
        ],
    )(x_pad, W, a2x128)


def _sc_body(h2_hbm, s_hbm, t_hbm, src_hbm, dst_hbm, zeros_hbm,
             part_hbm, ee_hbm,
             acc, s_tab, t_tab, ee_buf, src_all, dst_all,
             rows0, rows1, dstx0, dstx1, sem_g0, sem_g1, sem_s0, sem_s1):
    cid = lax.axis_index("c")
    sid = lax.axis_index("s")

    pltpu.sync_copy(s_hbm, s_tab)
    pltpu.sync_copy(t_hbm, t_tab)
    pltpu.sync_copy(src_hbm.at[sid], src_all)
    pltpu.sync_copy(dst_hbm.at[sid], dst_all)

    rows = (rows0, rows1)
    dstx = (dstx0, dstx1)
    sem_g = (sem_g0, sem_g1)
    sem_s = (sem_s0, sem_s1)

    row0 = sid * ROWS_PER_TILE
    pltpu.sync_copy(zeros_hbm, acc.at[pl.ds(row0, ROWS_PER_TILE)])
    plsc.subcore_barrier()

    def issue_gather(c, b):
        db = dstx[b]
        for k in range(CHUNK // 16):
            v = dst_all[c, pl.ds(k * 16, 16)]
            db[pl.ds(k * 16, 16)] = v + v + cid
        pltpu.async_copy(h2_hbm.at[db], rows[b], sem_g[b])

    def wait_gather(c, b):
        pltpu.make_async_copy(h2_hbm.at[dstx[b]], rows[b], sem_g[b]).wait()

    def issue_scat(c, b):
        pltpu.async_copy(rows[b], acc.at[src_all.at[c]], sem_s[b], add=True)

    def wait_scat(c, b):
        pltpu.make_async_copy(rows[b], acc.at[src_all.at[c]], sem_s[b]).wait()

    def compute(c, b):
        rb = rows[b]
        for k in range(CHUNK // 16):
            si = src_all[c, pl.ds(k * 16, 16)]
            di = dst_all[c, pl.ds(k * 16, 16)]
            sv = plsc.load_gather(s_tab, [si])
            tv = plsc.load_gather(t_tab, [di])
            e = sv + tv
            e = jnp.maximum(e, 0.2 * e)
            ev = jnp.exp(-e)
            cmod = c % HALF_CHUNKS
            ee_buf[pl.ds(cmod * CHUNK + k * 16, 16)] = ev
            for jj in range(16):
                eb = lax.gather(
                    ev, jnp.full((16, 1), jj, jnp.int32),
                    dimension_numbers=lax.GatherDimensionNumbers(
                        offset_dims=(), collapsed_slice_dims=(0,),
                        start_index_map=(0,)),
                    slice_sizes=(1,),
                    mode=lax.GatherScatterMode.PROMISE_IN_BOUNDS)
                j = k * 16 + jj
                for r in range(FH // 16):
                    rb[j, pl.ds(r * 16, 16)] = rb[j, pl.ds(r * 16, 16)] * eb

    issue_gather(0, 0)
    wait_gather(0, 0)
    issue_gather(1, 1)
    compute(0, 0)
    issue_scat(0, 0)

    def _pair(i, carry):
        for b, off in ((1, 1), (0, 2)):
            cc = 2 * i + off
            wait_gather(cc, b)
            wait_scat(cc - 1, 1 - b)
            issue_gather(cc + 1, 1 - b)
            compute(cc, b)
            issue_scat(cc, b)

            @pl.when(jnp.logical_and(cc == HALF_CHUNKS - 1, cid == 0))
            def _flush_first_half():
                pltpu.sync_copy(
                    ee_buf, ee_hbm.at[pl.ds(sid * EDGES_PER_TILE, EHALF)])
        return carry

    lax.fori_loop(0, (N_CHUNKS - 2) // 2, _pair, 0)
    cl = N_CHUNKS - 1
    wait_gather(cl, 1)
    wait_scat(cl - 1, 0)
    compute(cl, 1)
    issue_scat(cl, 1)
    wait_scat(cl, 1)
    plsc.subcore_barrier()

    pltpu.sync_copy(acc.at[pl.ds(row0, ROWS_PER_TILE)],
                    part_hbm.at[cid, pl.ds(row0, ROWS_PER_TILE)])

    @pl.when(cid == 1)
    def _flush_second_half():
        pltpu.sync_copy(
            ee_buf, ee_hbm.at[pl.ds(sid * EDGES_PER_TILE + EHALF, EHALF)])


def _stage2(h2, s, t, src, dst, zeros):
    mesh = plsc.VectorSubcoreMesh(
        core_axis_name="c", subcore_axis_name="s",
        num_cores=NUM_CORES, num_subcores=NUM_SUBCORES)
    f = functools.partial(
        pl.kernel,
        out_type=[
            jax.ShapeDtypeStruct((NUM_CORES, N_PAD, FH), jnp.float32),
            jax.ShapeDtypeStruct((N_EDGES,), jnp.float32),
        ],
        mesh=mesh,
        compiler_params=pltpu.CompilerParams(
            needs_layout_passes=False, use_tc_tiling_on_sc=False),
        scratch_types=[
            pltpu.VMEM_SHARED((N_PAD, FH), jnp.float32),
            pltpu.VMEM((N_PAD,), jnp.float32),
            pltpu.VMEM((N_PAD,), jnp.float32),
            pltpu.VMEM((EHALF,), jnp.float32),
            pltpu.VMEM((N_CHUNKS, CHUNK), jnp.int32),
            pltpu.VMEM((N_CHUNKS, CHUNK), jnp.int32),
            pltpu.VMEM((CHUNK, FH), jnp.float32),
            pltpu.VMEM((CHUNK, FH), jnp.float32),
            pltpu.VMEM((CHUNK,), jnp.int32),
            pltpu.VMEM((CHUNK,), jnp.int32),
            pltpu.SemaphoreType.DMA,
            pltpu.SemaphoreType.DMA,
            pltpu.SemaphoreType.DMA,
            pltpu.SemaphoreType.DMA,
        ],
    )(_sc_body)
    return f(h2, s, t, src, dst, zeros)


def _elu_body(p_ref, o_ref):
    z = jnp.concatenate([p_ref[0], p_ref[1]], axis=1)
    o_ref[...] = jnp.where(z > 0, z, jnp.exp(z) - 1.0)


def _stage3(part):
    bn3 = 400
    return pl.pallas_call(
        _elu_body,
        grid=(N_NODES // bn3,),
        in_specs=[pl.BlockSpec((2, bn3, FH), lambda i: (0, i, 0))],
        out_specs=pl.BlockSpec((bn3, F), lambda i: (i, 0)),
        out_shape=jax.ShapeDtypeStruct((N_NODES, F), jnp.float32),
    )(part)


def kernel(input, non_zero, W, a):
    x = jnp.asarray(input, jnp.float32)
    x_pad = jnp.pad(x, ((0, N_PAD - N_NODES), (0, 0)))
    a2 = jnp.asarray(a, jnp.float32).reshape(2, F)
    ei = jnp.asarray(non_zero, jnp.int32).reshape(2, NUM_SUBCORES, N_CHUNKS, CHUNK)
    src = ei[0]
    dst = ei[1]
    zeros = jnp.zeros((ROWS_PER_TILE, FH), jnp.float32)

    h, st = _stage1(x_pad, jnp.asarray(W, jnp.float32), a2)
    s = st[0]
    t = st[1]
    h2 = h.reshape(2 * N_PAD, FH)

    part, ee = _stage2(h2, s, t, src, dst, zeros)
    out = _stage3(part)
    return (out, ee)


# speedup vs baseline: 11.3312x; 1.2386x over previous
import functools

import jax
import jax.numpy as jnp
from jax import lax
from jax.experimental import pallas as pl
from jax.experimental.pallas import tpu as pltpu
from jax.experimental.pallas import tpu_sc as plsc

N_NODES = 10000
N_PAD = 10240
N_EDGES = 320000
F = 128
FH = F // 2

NUM_CORES = 2
NUM_SUBCORES = 16
EDGES_PER_TILE = N_EDGES // NUM_SUBCORES
CHUNK = 80
N_CHUNKS = EDGES_PER_TILE // CHUNK
HALF_CHUNKS = N_CHUNKS // 2
EHALF = EDGES_PER_TILE // 2
ROWS_PER_TILE = N_PAD // NUM_SUBCORES

BN = 256
GRID = N_PAD // BN


def _hst_body(x_ref, w_ref, a_ref, h_ref, st_ref):
    h = jnp.dot(x_ref[...], w_ref[...], preferred_element_type=jnp.float32)
    h_ref[...] = h
    a1 = a_ref[0, :]
    a2 = a_ref[1, :]
    s = jnp.dot(h, a1, preferred_element_type=jnp.float32)
    t = jnp.dot(h, a2, preferred_element_type=jnp.float32)
    pad = jnp.zeros((6, BN), jnp.float32)
    st_ref[...] = jnp.concatenate([s[None, :], t[None, :], pad], axis=0)


def _stage1(x_pad, W, a2x128):
    return pl.pallas_call(
        _hst_body,
        grid=(GRID,),
        in_specs=[
            pl.BlockSpec((BN, F), lambda i: (i, 0)),
            pl.BlockSpec((F, F), lambda i: (0, 0)),
            pl.BlockSpec((2, F), lambda i: (0, 0)),
        ],
        out_specs=[
            pl.BlockSpec((BN, F), lambda i: (i, 0)),
            pl.BlockSpec((8, BN), lambda i: (0, i)),
        ],
        out_shape=[
            jax.ShapeDtypeStruct((N_PAD, F), jnp.float32),
            jax.ShapeDtypeStruct((8, N_PAD), jnp.float32),
        ],
    )(x_pad, W, a2x128)


def _sc_body(h2_hbm, s_hbm, t_hbm, src_hbm, dst_hbm, zeros_hbm,
             part_hbm, ee_hbm,
             acc, s_tab, t_tab, ee_buf, src_all, dst_all,
             rows0, rows1, rows2, dstx0, dstx1, dstx2,
             sem_g0, sem_g1, sem_g2, sem_s0, sem_s1, sem_s2):
    cid = lax.axis_index("c")
    sid = lax.axis_index("s")

    row0 = sid * ROWS_PER_TILE
    d1 = pltpu.async_copy(s_hbm, s_tab, sem_g0)
    d2 = pltpu.async_copy(t_hbm, t_tab, sem_g1)
    d3 = pltpu.async_copy(src_hbm.at[sid], src_all, sem_g2)
    d4 = pltpu.async_copy(dst_hbm.at[sid], dst_all, sem_s0)
    d5 = pltpu.async_copy(zeros_hbm, acc.at[pl.ds(row0, ROWS_PER_TILE)], sem_s1)
    d1.wait(); d2.wait(); d3.wait(); d4.wait(); d5.wait()

    rows = (rows0, rows1, rows2)
    dstx = (dstx0, dstx1, dstx2)
    sem_g = (sem_g0, sem_g1, sem_g2)
    sem_s = (sem_s0, sem_s1, sem_s2)
    plsc.subcore_barrier()

    def issue_gather(c, b):
        db = dstx[b]
        for k in range(CHUNK // 16):
            v = dst_all[c, pl.ds(k * 16, 16)]
            db[pl.ds(k * 16, 16)] = v + v + cid
        pltpu.async_copy(h2_hbm.at[db], rows[b], sem_g[b])

    def wait_gather(c, b):
        pltpu.make_async_copy(h2_hbm.at[dstx[b]], rows[b], sem_g[b]).wait()

    def issue_scat(c, b):
        pltpu.async_copy(rows[b], acc.at[src_all.at[c]], sem_s[b], add=True)

    def wait_scat(c, b):
        pltpu.make_async_copy(rows[b], acc.at[src_all.at[c]], sem_s[b]).wait()

    def compute(c, b):
        rb = rows[b]
        for k in range(CHUNK // 16):
            si = src_all[c, pl.ds(k * 16, 16)]
            di = dst_all[c, pl.ds(k * 16, 16)]
            sv = plsc.load_gather(s_tab, [si])
            tv = plsc.load_gather(t_tab, [di])
            e = sv + tv
            e = jnp.maximum(e, 0.2 * e)
            ev = jnp.exp(-e)
            cmod = c % HALF_CHUNKS
            ee_buf[pl.ds(cmod * CHUNK + k * 16, 16)] = ev
            for jj in range(16):
                eb = lax.gather(
                    ev, jnp.full((16, 1), jj, jnp.int32),
                    dimension_numbers=lax.GatherDimensionNumbers(
                        offset_dims=(), collapsed_slice_dims=(0,),
                        start_index_map=(0,)),
                    slice_sizes=(1,),
                    mode=lax.GatherScatterMode.PROMISE_IN_BOUNDS)
                j = k * 16 + jj
                for r in range(FH // 16):
                    rb[j, pl.ds(r * 16, 16)] = rb[j, pl.ds(r * 16, 16)] * eb

    issue_gather(0, 0)
    issue_gather(1, 1)
    wait_gather(0, 0)
    compute(0, 0)
    issue_scat(0, 0)
    issue_gather(2, 2)
    wait_gather(1, 1)
    compute(1, 1)
    issue_scat(1, 1)
    wait_scat(0, 0)
    issue_gather(3, 0)

    def _triple(i, carry):
        for sub in range(3):
            cc = 3 * i + 2 + sub
            b = (2 + sub) % 3
            bn = (sub + 1) % 3
            wait_gather(cc, b)
            compute(cc, b)
            issue_scat(cc, b)
            wait_scat(cc - 1, bn)
            issue_gather(cc + 2, bn)

            @pl.when(jnp.logical_and(cc == HALF_CHUNKS - 1, cid == 0))
            def _flush_first_half():
                pltpu.sync_copy(
                    ee_buf, ee_hbm.at[pl.ds(sid * EDGES_PER_TILE, EHALF)])
        return carry

    lax.fori_loop(0, (N_CHUNKS - 4) // 3, _triple, 0)
    for cl in (N_CHUNKS - 2, N_CHUNKS - 1):
        b = cl % 3
        wait_gather(cl, b)
        compute(cl, b)
        issue_scat(cl, b)
    wait_scat(N_CHUNKS - 3, (N_CHUNKS - 3) % 3)
    wait_scat(N_CHUNKS - 2, (N_CHUNKS - 2) % 3)
    wait_scat(N_CHUNKS - 1, (N_CHUNKS - 1) % 3)
    plsc.subcore_barrier()

    pltpu.sync_copy(acc.at[pl.ds(row0, ROWS_PER_TILE)],
                    part_hbm.at[cid, pl.ds(row0, ROWS_PER_TILE)])

    @pl.when(cid == 1)
    def _flush_second_half():
        pltpu.sync_copy(
            ee_buf, ee_hbm.at[pl.ds(sid * EDGES_PER_TILE + EHALF, EHALF)])


def _stage2(h2, s, t, src, dst, zeros):
    mesh = plsc.VectorSubcoreMesh(
        core_axis_name="c", subcore_axis_name="s",
        num_cores=NUM_CORES, num_subcores=NUM_SUBCORES)
    f = functools.partial(
        pl.kernel,
        out_type=[
            jax.ShapeDtypeStruct((NUM_CORES, N_PAD, FH), jnp.float32),
            jax.ShapeDtypeStruct((N_EDGES,), jnp.float32),
        ],
        mesh=mesh,
        compiler_params=pltpu.CompilerParams(
            needs_layout_passes=False, use_tc_tiling_on_sc=False),
        scratch_types=[
            pltpu.VMEM_SHARED((N_PAD, FH), jnp.float32),
            pltpu.VMEM((N_PAD,), jnp.float32),
            pltpu.VMEM((N_PAD,), jnp.float32),
            pltpu.VMEM((EHALF,), jnp.float32),
            pltpu.VMEM((N_CHUNKS, CHUNK), jnp.int32),
            pltpu.VMEM((N_CHUNKS, CHUNK), jnp.int32),
            pltpu.VMEM((CHUNK, FH), jnp.float32),
            pltpu.VMEM((CHUNK, FH), jnp.float32),
            pltpu.VMEM((CHUNK, FH), jnp.float32),
            pltpu.VMEM((CHUNK,), jnp.int32),
            pltpu.VMEM((CHUNK,), jnp.int32),
            pltpu.VMEM((CHUNK,), jnp.int32),
            pltpu.SemaphoreType.DMA,
            pltpu.SemaphoreType.DMA,
            pltpu.SemaphoreType.DMA,
            pltpu.SemaphoreType.DMA,
            pltpu.SemaphoreType.DMA,
            pltpu.SemaphoreType.DMA,
        ],
    )(_sc_body)
    return f(h2, s, t, src, dst, zeros)


def _elu_body(p_ref, o_ref):
    z = jnp.concatenate([p_ref[0], p_ref[1]], axis=1)
    o_ref[...] = jnp.where(z > 0, z, jnp.exp(z) - 1.0)


def _stage3(part):
    bn3 = 400
    return pl.pallas_call(
        _elu_body,
        grid=(N_NODES // bn3,),
        in_specs=[pl.BlockSpec((2, bn3, FH), lambda i: (0, i, 0))],
        out_specs=pl.BlockSpec((bn3, F), lambda i: (i, 0)),
        out_shape=jax.ShapeDtypeStruct((N_NODES, F), jnp.float32),
    )(part)


def kernel(input, non_zero, W, a):
    x = jnp.asarray(input, jnp.float32)
    x_pad = jnp.pad(x, ((0, N_PAD - N_NODES), (0, 0)))
    a2 = jnp.asarray(a, jnp.float32).reshape(2, F)
    ei = jnp.asarray(non_zero, jnp.int32).reshape(2, NUM_SUBCORES, N_CHUNKS, CHUNK)
    src = ei[0]
    dst = ei[1]
    zeros = jnp.zeros((ROWS_PER_TILE, FH), jnp.float32)

    h, st = _stage1(x_pad, jnp.asarray(W, jnp.float32), a2)
    s = st[0]
    t = st[1]
    h2 = h.reshape(2 * N_PAD, FH)

    part, ee = _stage2(h2, s, t, src, dst, zeros)
    out = _stage3(part)
    return (out, ee)


# speedup vs baseline: 12.3084x; 1.0862x over previous
import functools

import jax
import jax.numpy as jnp
from jax import lax
from jax.experimental import pallas as pl
from jax.experimental.pallas import tpu as pltpu
from jax.experimental.pallas import tpu_sc as plsc

N_NODES = 10000
N_PAD = 10240
N_EDGES = 320000
F = 128
FH = F // 2

NUM_CORES = 2
NUM_SUBCORES = 16
EDGES_PER_TILE = N_EDGES // NUM_SUBCORES
CHUNK = 80
N_CHUNKS = EDGES_PER_TILE // CHUNK
HALF_CHUNKS = N_CHUNKS // 2
EHALF = EDGES_PER_TILE // 2
ROWS_PER_TILE = N_PAD // NUM_SUBCORES

BN = 256
GRID = N_PAD // BN


def _hst_body(x_ref, w_ref, a_ref, h_ref, st_ref):
    h = jnp.dot(x_ref[...], w_ref[...], preferred_element_type=jnp.float32)
    h_ref[...] = h
    a1 = a_ref[0, :]
    a2 = a_ref[1, :]
    s = jnp.dot(h, a1, preferred_element_type=jnp.float32)
    t = jnp.dot(h, a2, preferred_element_type=jnp.float32)
    pad = jnp.zeros((6, BN), jnp.float32)
    st_ref[...] = jnp.concatenate([s[None, :], t[None, :], pad], axis=0)


def _stage1(x_pad, W, a2x128):
    return pl.pallas_call(
        _hst_body,
        grid=(GRID,),
        in_specs=[
            pl.BlockSpec((BN, F), lambda i: (i, 0)),
            pl.BlockSpec((F, F), lambda i: (0, 0)),
            pl.BlockSpec((2, F), lambda i: (0, 0)),
        ],
        out_specs=[
            pl.BlockSpec((BN, F), lambda i: (i, 0)),
            pl.BlockSpec((8, BN), lambda i: (0, i)),
        ],
        out_shape=[
            jax.ShapeDtypeStruct((N_PAD, F), jnp.float32),
            jax.ShapeDtypeStruct((8, N_PAD), jnp.float32),
        ],
    )(x_pad, W, a2x128)


def _sc_body(h2_hbm, s_hbm, t_hbm, src_hbm, dst_hbm, zeros_hbm,
             out_hbm, ee_hbm,
             acc, s_tab, t_tab, ee_buf, src_all, dst_all,
             rows0, rows1, rows2, dstx0, dstx1, dstx2,
             sem_g0, sem_g1, sem_g2, sem_s0, sem_s1, sem_s2):
    cid = lax.axis_index("c")
    sid = lax.axis_index("s")

    row0 = sid * ROWS_PER_TILE
    d1 = pltpu.async_copy(s_hbm, s_tab, sem_g0)
    d2 = pltpu.async_copy(t_hbm, t_tab, sem_g1)
    d3 = pltpu.async_copy(src_hbm.at[sid], src_all, sem_g2)
    d4 = pltpu.async_copy(dst_hbm.at[sid], dst_all, sem_s0)
    d5 = pltpu.async_copy(zeros_hbm, acc.at[pl.ds(row0, ROWS_PER_TILE)], sem_s1)
    d1.wait(); d2.wait(); d3.wait(); d4.wait(); d5.wait()

    rows = (rows0, rows1, rows2)
    dstx = (dstx0, dstx1, dstx2)
    sem_g = (sem_g0, sem_g1, sem_g2)
    sem_s = (sem_s0, sem_s1, sem_s2)
    plsc.subcore_barrier()

    def issue_gather(c, b):
        db = dstx[b]
        for k in range(CHUNK // 16):
            v = dst_all[c, pl.ds(k * 16, 16)]
            db[pl.ds(k * 16, 16)] = v + v + cid
        pltpu.async_copy(h2_hbm.at[db], rows[b], sem_g[b])

    def wait_gather(c, b):
        pltpu.make_async_copy(h2_hbm.at[dstx[b]], rows[b], sem_g[b]).wait()

    def issue_scat(c, b):
        pltpu.async_copy(rows[b], acc.at[src_all.at[c]], sem_s[b], add=True)

    def wait_scat(c, b):
        pltpu.make_async_copy(rows[b], acc.at[src_all.at[c]], sem_s[b]).wait()

    def compute(c, b):
        rb = rows[b]
        for k in range(CHUNK // 16):
            si = src_all[c, pl.ds(k * 16, 16)]
            di = dst_all[c, pl.ds(k * 16, 16)]
            sv = plsc.load_gather(s_tab, [si])
            tv = plsc.load_gather(t_tab, [di])
            e = sv + tv
            e = jnp.maximum(e, 0.2 * e)
            ev = jnp.exp(-e)
            cmod = c % HALF_CHUNKS
            ee_buf[pl.ds(cmod * CHUNK + k * 16, 16)] = ev
            for jj in range(16):
                eb = lax.gather(
                    ev, jnp.full((16, 1), jj, jnp.int32),
                    dimension_numbers=lax.GatherDimensionNumbers(
                        offset_dims=(), collapsed_slice_dims=(0,),
                        start_index_map=(0,)),
                    slice_sizes=(1,),
                    mode=lax.GatherScatterMode.PROMISE_IN_BOUNDS)
                j = k * 16 + jj
                for r in range(FH // 16):
                    rb[j, pl.ds(r * 16, 16)] = rb[j, pl.ds(r * 16, 16)] * eb

    issue_gather(0, 0)
    issue_gather(1, 1)
    wait_gather(0, 0)
    compute(0, 0)
    issue_scat(0, 0)
    issue_gather(2, 2)
    wait_gather(1, 1)
    compute(1, 1)
    issue_scat(1, 1)
    wait_scat(0, 0)
    issue_gather(3, 0)

    def _triple(i, carry):
        for sub in range(3):
            cc = 3 * i + 2 + sub
            b = (2 + sub) % 3
            bn = (sub + 1) % 3
            wait_gather(cc, b)
            compute(cc, b)
            issue_scat(cc, b)
            wait_scat(cc - 1, bn)
            issue_gather(cc + 2, bn)

            @pl.when(jnp.logical_and(cc == HALF_CHUNKS - 1, cid == 0))
            def _flush_first_half():
                pltpu.sync_copy(
                    ee_buf, ee_hbm.at[pl.ds(sid * EDGES_PER_TILE, EHALF)])
        return carry

    lax.fori_loop(0, (N_CHUNKS - 4) // 3, _triple, 0)
    for cl in (N_CHUNKS - 2, N_CHUNKS - 1):
        b = cl % 3
        wait_gather(cl, b)
        compute(cl, b)
        issue_scat(cl, b)
    wait_scat(N_CHUNKS - 3, (N_CHUNKS - 3) % 3)
    wait_scat(N_CHUNKS - 2, (N_CHUNKS - 2) % 3)
    wait_scat(N_CHUNKS - 1, (N_CHUNKS - 1) % 3)
    plsc.subcore_barrier()

    @pl.when(cid == 1)
    def _flush_second_half():
        pltpu.sync_copy(
            ee_buf, ee_hbm.at[pl.ds(sid * EDGES_PER_TILE + EHALF, EHALF)])

    n_batches = ROWS_PER_TILE // CHUNK

    def _elu_batch(k):
        r0 = row0 + k * CHUNK
        pltpu.sync_copy(acc.at[pl.ds(r0, CHUNK)], rows0)

        def _elu_row(j, carry):
            for r in range(FH // 16):
                z = rows0[j, pl.ds(r * 16, 16)]
                rows0[j, pl.ds(r * 16, 16)] = jnp.where(
                    z > 0, z, jnp.exp(z) - 1.0)
            return carry

        lax.fori_loop(0, CHUNK, _elu_row, 0)
        pltpu.sync_copy(rows0,
                        out_hbm.at[pl.ds(r0, CHUNK), pl.ds(cid * FH, FH)])

    for k in range(5):
        _elu_batch(k)
    @pl.when(sid < NUM_SUBCORES - 1)
    def _tail_batches():
        for k in range(5, n_batches):
            _elu_batch(k)


def _stage2(h2, s, t, src, dst, zeros):
    mesh = plsc.VectorSubcoreMesh(
        core_axis_name="c", subcore_axis_name="s",
        num_cores=NUM_CORES, num_subcores=NUM_SUBCORES)
    f = functools.partial(
        pl.kernel,
        out_type=[
            jax.ShapeDtypeStruct((N_NODES, F), jnp.float32),
            jax.ShapeDtypeStruct((N_EDGES,), jnp.float32),
        ],
        mesh=mesh,
        compiler_params=pltpu.CompilerParams(
            needs_layout_passes=False, use_tc_tiling_on_sc=False),
        scratch_types=[
            pltpu.VMEM_SHARED((N_PAD, FH), jnp.float32),
            pltpu.VMEM((N_PAD,), jnp.float32),
            pltpu.VMEM((N_PAD,), jnp.float32),
            pltpu.VMEM((EHALF,), jnp.float32),
            pltpu.VMEM((N_CHUNKS, CHUNK), jnp.int32),
            pltpu.VMEM((N_CHUNKS, CHUNK), jnp.int32),
            pltpu.VMEM((CHUNK, FH), jnp.float32),
            pltpu.VMEM((CHUNK, FH), jnp.float32),
            pltpu.VMEM((CHUNK, FH), jnp.float32),
            pltpu.VMEM((CHUNK,), jnp.int32),
            pltpu.VMEM((CHUNK,), jnp.int32),
            pltpu.VMEM((CHUNK,), jnp.int32),
            pltpu.SemaphoreType.DMA,
            pltpu.SemaphoreType.DMA,
            pltpu.SemaphoreType.DMA,
            pltpu.SemaphoreType.DMA,
            pltpu.SemaphoreType.DMA,
            pltpu.SemaphoreType.DMA,
        ],
    )(_sc_body)
    return f(h2, s, t, src, dst, zeros)


def kernel(input, non_zero, W, a):
    x = jnp.asarray(input, jnp.float32)
    a2 = jnp.asarray(a, jnp.float32).reshape(2, F)
    ei = jnp.asarray(non_zero, jnp.int32).reshape(2, NUM_SUBCORES, N_CHUNKS, CHUNK)
    src = ei[0]
    dst = ei[1]
    zeros = jnp.zeros((ROWS_PER_TILE, FH), jnp.float32)

    h, st = _stage1(x, jnp.asarray(W, jnp.float32), a2)
    s = st[0]
    t = st[1]
    h2 = h.reshape(2 * N_PAD, FH)

    out, ee = _stage2(h2, s, t, src, dst, zeros)
    return (out, ee)
